# Initial kernel scaffold; baseline (speedup 1.0000x reference)
import jax
import jax.numpy as jnp
from jax.experimental import pallas as pl


def kernel(x, edge_index, edge_trust_score, edge_query_embedding, lw1, lb1, e1w1, e1b1, e1w2, e1b2, lw2, lb2, e2w1, e2b1, e2w2, e2b2, pw1, pb1, pw2, pb2):
    raise NotImplementedError("write your pallas kernel here")


# speedup vs baseline: 2.3354x; 2.3354x over previous
import functools

import jax
import jax.numpy as jnp
from jax import lax
from jax.experimental import pallas as pl
from jax.experimental.pallas import tpu as pltpu
from jax.experimental.pallas import tpu_sc as plsc

N = 10000
E = 320000
EMB = 128
EF2 = 258

NC = 2
NS = 16
NW = NC * NS
EPW = E // NW
C = 80
NCH = EPW // C
NP = 10240
ZR = NP // NS

_BE = 2000
_F32 = jnp.float32


def _edge_mlp_body(t_ref, q_ref, wq1_ref, wt1_ref, bc1_ref, v1_ref, b1_ref,
                   wq2_ref, wt2_ref, bc2_ref, v2_ref, b2_ref, o1_ref, o2_ref):
    q = q_ref[...]
    t = t_ref[...]
    rh1 = jnp.maximum(
        jnp.dot(q, wq1_ref[...], preferred_element_type=_F32)
        + t * wt1_ref[...] + bc1_ref[...], 0.0)
    o1_ref[...] = jnp.dot(rh1, v1_ref[...], preferred_element_type=_F32) + b1_ref[...]
    rh2 = jnp.maximum(
        jnp.dot(q, wq2_ref[...], preferred_element_type=_F32)
        + t * wt2_ref[...] + bc2_ref[...], 0.0)
    o2_ref[...] = jnp.dot(rh2, v2_ref[...], preferred_element_type=_F32) + b2_ref[...]


def _full2d(shape):
    return pl.BlockSpec(shape, lambda i: (0, 0))


_edge_mlp = pl.pallas_call(
    _edge_mlp_body,
    grid=(E // _BE,),
    in_specs=[
        pl.BlockSpec((_BE, 1), lambda i: (i, 0)),
        pl.BlockSpec((_BE, EMB), lambda i: (i, 0)),
        _full2d((EMB, EF2)), _full2d((1, EF2)), _full2d((1, EF2)),
        _full2d((EF2, EMB)), _full2d((1, EMB)),
        _full2d((EMB, EF2)), _full2d((1, EF2)), _full2d((1, EF2)),
        _full2d((EF2, EMB)), _full2d((1, EMB)),
    ],
    out_specs=[pl.BlockSpec((_BE, EMB), lambda i: (i, 0)),
               pl.BlockSpec((_BE, EMB), lambda i: (i, 0))],
    out_shape=[jax.ShapeDtypeStruct((E, EMB), _F32),
               jax.ShapeDtypeStruct((E, EMB), _F32)],
)


_sc_mesh = plsc.VectorSubcoreMesh(core_axis_name="c", subcore_axis_name="s")


@functools.partial(
    pl.kernel,
    out_type=jax.ShapeDtypeStruct((NC, NP, EMB), _F32),
    mesh=_sc_mesh,
    scratch_types=[
        pltpu.VMEM((C,), jnp.int32),
        pltpu.VMEM((C,), jnp.int32),
        pltpu.VMEM((C, EMB), _F32),
        pltpu.VMEM((C, EMB), _F32),
        pltpu.VMEM_SHARED((NP, EMB), _F32),
        pltpu.SemaphoreType.DMA,
    ],
)
def _sc_scatter(vals, table, src, dst, zeros, out, srci, dsti, vbuf, rbuf, acc, sem):
    c = lax.axis_index("c")
    s = lax.axis_index("s")
    pltpu.sync_copy(zeros.at[pl.ds(s * ZR, ZR)], acc.at[pl.ds(s * ZR, ZR)])
    plsc.subcore_barrier()
    base = (c * NS + s) * EPW

    def chunk(i, carry):
        off = base + i * C
        pltpu.sync_copy(src.at[pl.ds(off, C)], srci)
        pltpu.sync_copy(dst.at[pl.ds(off, C)], dsti)
        pltpu.sync_copy(vals.at[pl.ds(off, C)], vbuf)
        pltpu.async_copy(table.at[srci], rbuf, sem).wait()
        pltpu.sync_copy(vbuf, acc.at[dsti], add=True)
        pltpu.sync_copy(rbuf, acc.at[dsti], add=True)
        return carry

    lax.fori_loop(0, NCH, chunk, 0)
    plsc.subcore_barrier()
    pltpu.sync_copy(acc.at[pl.ds(s * ZR, ZR)], out.at[c, pl.ds(s * ZR, ZR)])


@functools.partial(
    pl.kernel,
    out_type=jax.ShapeDtypeStruct((E, EMB), _F32),
    mesh=_sc_mesh,
    scratch_types=[
        pltpu.VMEM((C,), jnp.int32),
        pltpu.VMEM((C,), jnp.int32),
        pltpu.VMEM((C, EMB), _F32),
        pltpu.VMEM((C, EMB), _F32),
        pltpu.SemaphoreType.DMA,
        pltpu.SemaphoreType.DMA,
    ],
)
def _sc_gather(asrc, adst, src, dst, out, srci, dsti, abuf, bbuf, sem_a, sem_b):
    c = lax.axis_index("c")
    s = lax.axis_index("s")
    base = (c * NS + s) * EPW

    def chunk(i, carry):
        off = base + i * C
        pltpu.sync_copy(src.at[pl.ds(off, C)], srci)
        pltpu.sync_copy(dst.at[pl.ds(off, C)], dsti)
        cp_a = pltpu.async_copy(asrc.at[srci], abuf, sem_a)
        cp_b = pltpu.async_copy(adst.at[dsti], bbuf, sem_b)
        cp_a.wait()
        cp_b.wait()

        def row(r, cc):
            for k in range(EMB // 16):
                sl = pl.ds(k * 16, 16)
                abuf[r, sl] = abuf[r, sl] + bbuf[r, sl]
            return cc

        lax.fori_loop(0, C, row, 0)
        pltpu.sync_copy(abuf, out.at[pl.ds(off, C)])
        return carry

    lax.fori_loop(0, NCH, chunk, 0)


def _node1_body(p_ref, w_ref, b_ref, o_ref):
    a = p_ref[0] + p_ref[1]
    o_ref[...] = jnp.maximum(
        jnp.dot(a, w_ref[...], preferred_element_type=_F32) + b_ref[...], 0.0)


_node1 = pl.pallas_call(
    _node1_body,
    out_shape=jax.ShapeDtypeStruct((NP, EMB), _F32),
)


def _node2_body(p_ref, w_ref, b_ref, wa_ref, wb_ref, pb_ref, oas_ref, oad_ref):
    a = p_ref[0] + p_ref[1]
    h2 = jnp.maximum(
        jnp.dot(a, w_ref[...], preferred_element_type=_F32) + b_ref[...], 0.0)
    oas_ref[...] = jnp.dot(h2, wa_ref[...], preferred_element_type=_F32)
    oad_ref[...] = jnp.dot(h2, wb_ref[...], preferred_element_type=_F32) + pb_ref[...]


_node2 = pl.pallas_call(
    _node2_body,
    out_shape=[jax.ShapeDtypeStruct((NP, EMB), _F32),
               jax.ShapeDtypeStruct((NP, EMB), _F32)],
)


def _pred_body(g_ref, q_ref, wc_ref, w2_ref, b2_ref, o_ref):
    p = jnp.maximum(
        g_ref[...] + jnp.dot(q_ref[...], wc_ref[...], preferred_element_type=_F32),
        0.0)
    z = jnp.sum(p * w2_ref[...], axis=1, keepdims=True) + b2_ref[...]
    o_ref[...] = jax.nn.sigmoid(z)


_pred = pl.pallas_call(
    _pred_body,
    grid=(E // _BE,),
    in_specs=[
        pl.BlockSpec((_BE, EMB), lambda i: (i, 0)),
        pl.BlockSpec((_BE, EMB), lambda i: (i, 0)),
        _full2d((EMB, EMB)), _full2d((1, EMB)), _full2d((1, 1)),
    ],
    out_specs=pl.BlockSpec((_BE, 1), lambda i: (i, 0)),
    out_shape=jax.ShapeDtypeStruct((E, 1), _F32),
)


def kernel(x, edge_index, edge_trust_score, edge_query_embedding,
           lw1, lb1, e1w1, e1b1, e1w2, e1b2,
           lw2, lb2, e2w1, e2b1, e2w2, e2b2,
           pw1, pb1, pw2, pb2):
    src = edge_index[0].astype(jnp.int32)
    dst = edge_index[1].astype(jnp.int32)
    t = edge_trust_score
    q = edge_query_embedding

    et1, et2 = _edge_mlp(
        t, q,
        e1w1[:, 1:].T, e1w1[:, :1].T, e1b1[None], e1w2.T, e1b2[None],
        e2w1[:, 1:].T, e2w1[:, :1].T, e2b1[None], e2w2.T, e2b2[None],
    )

    zeros = jnp.zeros((NP, EMB), _F32)
    p1 = _sc_scatter(et1, x, src, dst, zeros)
    h1 = _node1(p1, lw1.T, lb1[None])
    p2 = _sc_scatter(et2, h1, src, dst, zeros)
    asrc, adst = _node2(p2, lw2.T, lb2[None],
                        pw1[:, :EMB].T, pw1[:, EMB:2 * EMB].T, pb1[None])
    gsum = _sc_gather(asrc, adst, src, dst)
    out = _pred(gsum, q, pw1[:, 2 * EMB:].T, pw2, pb2[None])
    return out
